# baseline (speedup 1.0000x reference)
import jax
import jax.numpy as jnp
from jax import lax
from jax.experimental import pallas as pl
from jax.experimental.pallas import tpu as pltpu
from jax.experimental.pallas import tpu_sc as plsc

N = 10000
E = 320000
D = 128
P = 4
NPAD = 10240
HALF = NPAD // 2
NB = 2 * P
NTILES = 32
EPT = E // NTILES
ECH = 2000
NECH = EPT // ECH
CAP = 10240
CHUNK = 128
STG = CHUNK + 16
DROW = 64
ACCR = HALF + 2 * DROW
CSIZE = P * NPAD + 256
PKS = 16384


def _sc_body(x_ref, src_ref, dst_ref, pmap_ref,
             ssum_ref, cnt_ref, bkt_ref,
             pmap_v,
             st0, st1, st2, st3, st4, st5, st6, st7,
             esrc_a, esrc_b, edst_a, edst_b,
             rowbuf0, rowbuf1, pkbuf0, pkbuf1,
             stage0, stage1, ones_v, z2d, z1d,
             sem_ea, sem_eb, sem_pk0, sem_pk1,
             sem_g0, sem_g1, sem_s0, sem_s1, sem_c0, sem_c1,
             acc, cntacc):
    sts = [st0, st1, st2, st3, st4, st5, st6, st7]
    esrcs, edsts = [esrc_a, esrc_b], [edst_a, edst_b]
    esems = [sem_ea, sem_eb]
    rowbufs, pkbufs, stages = [rowbuf0, rowbuf1], [pkbuf0, pkbuf1], [stage0, stage1]
    sem_pk, sem_g, sem_s, sem_c = ([sem_pk0, sem_pk1], [sem_g0, sem_g1],
                                   [sem_s0, sem_s1], [sem_c0, sem_c1])
    cid = lax.axis_index("c")
    sid = lax.axis_index("s")
    wid = cid * 16 + sid
    ebase = pl.multiple_of(wid * EPT, 8)
    lanes = lax.iota(jnp.int32, 16)

    pltpu.sync_copy(pmap_ref, pmap_v)

    ones16 = jnp.ones((16,), jnp.float32)
    for k in range(CHUNK // 16):
        ones_v[pl.ds(k * 16, 16)] = ones16

    z16f = jnp.zeros((16,), jnp.float32)

    def _zero_z2d(j, _):
        r = j // (D // 16)
        c = (j % (D // 16)) * 16
        z2d[r, pl.ds(c, 16)] = z16f
        return 0
    lax.fori_loop(0, CHUNK * (D // 16), _zero_z2d, 0)

    def _zero_z1d(j, _):
        z1d[pl.ds(j * 16, 16)] = z16f
        return 0
    lax.fori_loop(0, (CSIZE // 16) // 16, _zero_z1d, 0)

    pltpu.sync_copy(z1d, cntacc.at[pl.ds(
        pl.multiple_of(sid * (CSIZE // 16), 8), CSIZE // 16)])

    dum_src = (wid * 313 + lanes * 13) % N
    dum_loc = HALF + (wid * 16 + lanes) % DROW
    dum_pk = dum_src * PKS + dum_loc

    zero8 = tuple(jnp.int32(0) for _ in range(NB))
    cur = zero8
    fil = zero8
    pltpu.async_copy(src_ref.at[pl.ds(ebase, ECH)], esrcs[0], esems[0])
    pltpu.async_copy(dst_ref.at[pl.ds(ebase, ECH)], edsts[0], esems[0])
    for a in range(NECH):
        pa = a % 2
        pltpu.make_async_copy(src_ref.at[pl.ds(ebase + a * ECH, ECH)],
                              esrcs[pa], esems[pa]).wait()
        pltpu.make_async_copy(dst_ref.at[pl.ds(ebase + a * ECH, ECH)],
                              edsts[pa], esems[pa]).wait()
        if a + 1 < NECH:
            pn = (a + 1) % 2
            pltpu.async_copy(src_ref.at[pl.ds(ebase + (a + 1) * ECH, ECH)],
                             esrcs[pn], esems[pn])
            pltpu.async_copy(dst_ref.at[pl.ds(ebase + (a + 1) * ECH, ECH)],
                             edsts[pn], esems[pn])
        esrc_v = esrcs[pa]
        edst_v = edsts[pa]

        def _bucket(j, state):
            cur, fil = state
            s16 = esrc_v[pl.ds(j * 16, 16)]
            d16 = edst_v[pl.ds(j * 16, 16)]
            rh16 = (d16 >= HALF).astype(jnp.int32)
            loc16 = d16 - rh16 * HALF
            pk16 = s16 * PKS + loc16
            part = plsc.load_gather(pmap_v, [s16])
            ncur = []
            nfil = []
            for b in range(NB):
                s, rh = b // 2, b % 2
                m = (part == s) & (rh16 == rh)
                n = jnp.sum(m.astype(jnp.int32))
                plsc.store_compressed(sts[b].at[pl.ds(fil[b], 16)],
                                      pk16, mask=m)
                c2 = cur[b] + n
                f2 = fil[b] + n
                flushed = c2 - f2

                @pl.when(f2 >= CHUNK)
                def _():
                    off = pl.multiple_of(
                        (b * NTILES + wid) * CAP + flushed, 8)
                    pltpu.sync_copy(sts[b].at[pl.ds(0, CHUNK)],
                                    bkt_ref.at[pl.ds(off, CHUNK)])
                    rem = sts[b][pl.ds(CHUNK, 16)]
                    sts[b][pl.ds(0, 16)] = rem

                f2 = jnp.where(f2 >= CHUNK, f2 - CHUNK, f2)
                ncur.append(c2)
                nfil.append(f2)
            return tuple(ncur), tuple(nfil)
        cur, fil = lax.fori_loop(0, ECH // 16, _bucket, (cur, fil))

    for b in range(NB):
        for k in range(CHUNK // 16):
            pos = k * 16 + lanes
            stv = sts[b][pl.ds(k * 16, 16)]
            sts[b][pl.ds(k * 16, 16)] = jnp.where(pos >= fil[b], dum_pk, stv)
        flushed = cur[b] - fil[b]

        @pl.when(fil[b] > 0)
        def _():
            off = pl.multiple_of((b * NTILES + wid) * CAP + flushed, 8)
            pltpu.sync_copy(sts[b].at[pl.ds(0, CHUNK)],
                            bkt_ref.at[pl.ds(off, CHUNK)])

    nch = [(cur[b] + (CHUNK - 1)) // CHUNK for b in range(NB)]

    for b in range(NB):
        s, rh = b // 2, b % 2
        srows = ACCR // 16
        pltpu.sync_copy(z2d, acc.at[pl.ds(sid * srows, CHUNK), :])
        pltpu.sync_copy(z2d, acc.at[pl.ds(sid * srows + CHUNK, CHUNK), :])
        pltpu.sync_copy(z2d.at[pl.ds(0, srows - 2 * CHUNK), :],
                        acc.at[pl.ds(sid * srows + 2 * CHUNK,
                                     srows - 2 * CHUNK), :])
        plsc.subcore_barrier()

        cbase = s * NPAD + rh * HALF
        bkbase = (b * NTILES + wid) * CAP

        def _pk_copy(c, p):
            off = pl.multiple_of(bkbase + c * CHUNK, 8)
            return pltpu.make_async_copy(bkt_ref.at[pl.ds(off, CHUNK)],
                                         pkbufs[p], sem_pk[p])

        for p in range(2):
            @pl.when(p < nch[b])
            def _(p=p):
                off = pl.multiple_of(bkbase + p * CHUNK, 8)
                pltpu.async_copy(bkt_ref.at[pl.ds(off, CHUNK)],
                                 pkbufs[p], sem_pk[p])

        def _pair(i, _):
            for p in range(2):
                c = 2 * i + p

                @pl.when(c < nch[b])
                def _(c=c, p=p):
                    @pl.when(c >= 2)
                    def _():
                        pltpu.make_async_copy(
                            rowbufs[p], acc.at[stages[p].at[0]],
                            sem_s[p]).wait()
                        pltpu.make_async_copy(
                            ones_v, cntacc.at[stages[p].at[2]],
                            sem_c[p]).wait()
                    _pk_copy(c, p).wait()
                    for k in range(CHUNK // 16):
                        pk = pkbufs[p][pl.ds(k * 16, 16)]
                        loc = pk & (PKS - 1)
                        stages[p][0, pl.ds(k * 16, 16)] = loc
                        stages[p][1, pl.ds(k * 16, 16)] = pk >> 14
                        stages[p][2, pl.ds(k * 16, 16)] = jnp.where(
                            loc >= HALF, P * NPAD + (loc - HALF), cbase + loc)

                    @pl.when(c + 2 < nch[b])
                    def _():
                        off2 = pl.multiple_of(bkbase + (c + 2) * CHUNK, 8)
                        pltpu.async_copy(bkt_ref.at[pl.ds(off2, CHUNK)],
                                         pkbufs[p], sem_pk[p])
                    pltpu.async_copy(x_ref.at[stages[p].at[1]],
                                     rowbufs[p], sem_g[p])
            for p in range(2):
                c = 2 * i + p

                @pl.when(c < nch[b])
                def _(c=c, p=p):
                    pltpu.make_async_copy(x_ref.at[stages[p].at[1]],
                                          rowbufs[p], sem_g[p]).wait()
                    pltpu.async_copy(rowbufs[p], acc.at[stages[p].at[0]],
                                     sem_s[p], add=True)
                    pltpu.async_copy(ones_v, cntacc.at[stages[p].at[2]],
                                     sem_c[p], add=True)
            return 0
        lax.fori_loop(0, (nch[b] + 1) // 2, _pair, 0)
        for p in range(2):
            @pl.when(nch[b] > p)
            def _(p=p):
                pltpu.make_async_copy(rowbufs[p], acc.at[stages[p].at[0]],
                                      sem_s[p]).wait()
                pltpu.make_async_copy(ones_v, cntacc.at[stages[p].at[2]],
                                      sem_c[p]).wait()
        plsc.subcore_barrier()

        pltpu.sync_copy(
            acc.at[pl.ds(sid * (HALF // 16), HALF // 16), :],
            ssum_ref.at[2 * s + cid,
                        pl.ds(rh * HALF + sid * (HALF // 16), HALF // 16), :])
        plsc.subcore_barrier()

    coff = pl.multiple_of(cid * (P * NPAD) + sid * (P * NPAD // 16), 8)
    pltpu.sync_copy(
        cntacc.at[pl.ds(pl.multiple_of(sid * (P * NPAD // 16), 8),
                        P * NPAD // 16)],
        cnt_ref.at[pl.ds(coff, P * NPAD // 16)])


@jax.jit
def _sc_aggregate(x, src, dst, p_map):
    mesh = plsc.VectorSubcoreMesh(core_axis_name="c", subcore_axis_name="s")
    f = pl.kernel(
        _sc_body,
        out_type=(
            jax.ShapeDtypeStruct((NB, NPAD, D), jnp.float32),
            jax.ShapeDtypeStruct((2 * P * NPAD,), jnp.float32),
            jax.ShapeDtypeStruct((NB * NTILES * CAP,), jnp.int32),
        ),
        mesh=mesh,
        compiler_params=pltpu.CompilerParams(needs_layout_passes=False),
        scratch_types=[
            pltpu.VMEM((N,), jnp.int32),
            pltpu.VMEM((STG,), jnp.int32),
            pltpu.VMEM((STG,), jnp.int32),
            pltpu.VMEM((STG,), jnp.int32),
            pltpu.VMEM((STG,), jnp.int32),
            pltpu.VMEM((STG,), jnp.int32),
            pltpu.VMEM((STG,), jnp.int32),
            pltpu.VMEM((STG,), jnp.int32),
            pltpu.VMEM((STG,), jnp.int32),
            pltpu.VMEM((ECH,), jnp.int32),
            pltpu.VMEM((ECH,), jnp.int32),
            pltpu.VMEM((ECH,), jnp.int32),
            pltpu.VMEM((ECH,), jnp.int32),
            pltpu.VMEM((CHUNK, D), jnp.float32),
            pltpu.VMEM((CHUNK, D), jnp.float32),
            pltpu.VMEM((CHUNK,), jnp.int32),
            pltpu.VMEM((CHUNK,), jnp.int32),
            pltpu.VMEM((3, CHUNK), jnp.int32),
            pltpu.VMEM((3, CHUNK), jnp.int32),
            pltpu.VMEM((CHUNK,), jnp.float32),
            pltpu.VMEM((CHUNK, D), jnp.float32),
            pltpu.VMEM((CSIZE // 16,), jnp.float32),
            pltpu.SemaphoreType.DMA,
            pltpu.SemaphoreType.DMA,
            pltpu.SemaphoreType.DMA,
            pltpu.SemaphoreType.DMA,
            pltpu.SemaphoreType.DMA,
            pltpu.SemaphoreType.DMA,
            pltpu.SemaphoreType.DMA,
            pltpu.SemaphoreType.DMA,
            pltpu.SemaphoreType.DMA,
            pltpu.SemaphoreType.DMA,
            pltpu.VMEM_SHARED((ACCR, D), jnp.float32),
            pltpu.VMEM_SHARED((CSIZE,), jnp.float32),
        ],
    )
    return f(x, src, dst, p_map)


TBLK = 1000


def _tc_body(ssum_ref, cnt_ref, x_ref, pmap_ref, ws_ref, wn_ref, b_ref,
             out_ref):
    xb = x_ref[...]
    out = jnp.zeros_like(out_ref)
    for s in range(P):
        ssb = ssum_ref[2 * s] + ssum_ref[2 * s + 1]
        c = cnt_ref[:, s] + cnt_ref[:, P + s]
        inv = 1.0 / jnp.maximum(c, 1.0)
        mean = ssb * inv[:, None]
        out += lax.dot_general(mean, wn_ref[s], (((1,), (0,)), ((), ())),
                               preferred_element_type=jnp.float32,
                               precision=lax.Precision.HIGHEST)
    pm = pmap_ref[...]
    for t in range(P):
        sel = (pm == t).astype(jnp.float32)
        h = lax.dot_general(xb, ws_ref[t], (((1,), (0,)), ((), ())),
                            preferred_element_type=jnp.float32,
                            precision=lax.Precision.HIGHEST)
        out += sel * (h + b_ref[t][None, :])
    out_ref[...] = out


@jax.jit
def _tc_merge(ssum, cnt_t, x, pmap_t, W_self, W_neigh, b_pad):
    grid = N // TBLK
    return pl.pallas_call(
        _tc_body,
        grid=(grid,),
        in_specs=[
            pl.BlockSpec((NB, TBLK, D), lambda i: (0, i, 0)),
            pl.BlockSpec((TBLK, 2 * P), lambda i: (i, 0)),
            pl.BlockSpec((TBLK, D), lambda i: (i, 0)),
            pl.BlockSpec((TBLK, 1), lambda i: (i, 0)),
            pl.BlockSpec((P, D, D), lambda i: (0, 0, 0)),
            pl.BlockSpec((P, D, D), lambda i: (0, 0, 0)),
            pl.BlockSpec((2 * P, D), lambda i: (0, 0)),
        ],
        out_specs=pl.BlockSpec((TBLK, D), lambda i: (i, 0)),
        out_shape=jax.ShapeDtypeStruct((N, D), jnp.float32),
    )(ssum, cnt_t, x, pmap_t, W_self, W_neigh, b_pad)


def kernel(x, edge_index, p_map, W_self, W_neigh, b):
    src = edge_index[0]
    dst = edge_index[1]
    ssum, cnt, _ = _sc_aggregate(x, src, dst, p_map)
    cnt_t = cnt.reshape(2 * P, NPAD).T
    pmap_t = p_map[:, None]
    b_pad = jnp.pad(b, ((0, P), (0, 0)))
    return _tc_merge(ssum, cnt_t, x, pmap_t, W_self, W_neigh, b_pad)


# speedup vs baseline: 1.0728x; 1.0728x over previous
import jax
import jax.numpy as jnp
from jax import lax
from jax.experimental import pallas as pl
from jax.experimental.pallas import tpu as pltpu
from jax.experimental.pallas import tpu_sc as plsc

N = 10000
E = 320000
D = 128
P = 4
NPAD = 10240
HALF = NPAD // 2
NB = 2 * P
NTILES = 32
EPT = E // NTILES
ECH = 2000
NECH = EPT // ECH
CAP = 10240
CHUNK = 128
STG = CHUNK + 16
DROW = 64
ACCR = HALF + 2 * DROW
CSIZE = P * NPAD + 256
PKS = 16384


def _sc_body(x_ref, src_ref, dst_ref, pmap_ref,
             ssum_ref, cnt_ref, bkt_ref,
             pmap_v,
             st0, st1, st2, st3, st4, st5, st6, st7,
             esrc_a, esrc_b, edst_a, edst_b,
             rowbuf0, rowbuf1, pkbuf0, pkbuf1,
             stage0, stage1, ones_v, z2d, z1d,
             sem_ea, sem_eb, sem_pk0, sem_pk1,
             sem_g0, sem_g1, sem_s0, sem_s1, sem_c0, sem_c1,
             acc, cntacc):
    sts = [st0, st1, st2, st3, st4, st5, st6, st7]
    esrcs, edsts = [esrc_a, esrc_b], [edst_a, edst_b]
    esems = [sem_ea, sem_eb]
    rowbufs, pkbufs, stages = [rowbuf0, rowbuf1], [pkbuf0, pkbuf1], [stage0, stage1]
    sem_pk, sem_g, sem_s, sem_c = ([sem_pk0, sem_pk1], [sem_g0, sem_g1],
                                   [sem_s0, sem_s1], [sem_c0, sem_c1])
    cid = lax.axis_index("c")
    sid = lax.axis_index("s")
    wid = cid * 16 + sid
    ebase = pl.multiple_of(wid * EPT, 8)
    lanes = lax.iota(jnp.int32, 16)

    pltpu.sync_copy(pmap_ref, pmap_v)

    ones16 = jnp.ones((16,), jnp.float32)
    for k in range(CHUNK // 16):
        ones_v[pl.ds(k * 16, 16)] = ones16

    z16f = jnp.zeros((16,), jnp.float32)

    def _zero_z2d(j, _):
        r = j // (D // 16)
        c = (j % (D // 16)) * 16
        z2d[r, pl.ds(c, 16)] = z16f
        return 0
    lax.fori_loop(0, CHUNK * (D // 16), _zero_z2d, 0)

    def _zero_z1d(j, _):
        z1d[pl.ds(j * 16, 16)] = z16f
        return 0
    lax.fori_loop(0, (CSIZE // 16) // 16, _zero_z1d, 0)

    pltpu.sync_copy(z1d, cntacc.at[pl.ds(
        pl.multiple_of(sid * (CSIZE // 16), 8), CSIZE // 16)])

    dum_src = (wid * 313 + lanes * 13) % N
    dum_loc = HALF + (wid * 16 + lanes) % DROW
    dum_pk = dum_src * PKS + dum_loc

    zero8 = tuple(jnp.int32(0) for _ in range(NB))
    cur = zero8
    fil = zero8
    pltpu.async_copy(src_ref.at[pl.ds(ebase, ECH)], esrcs[0], esems[0])
    pltpu.async_copy(dst_ref.at[pl.ds(ebase, ECH)], edsts[0], esems[0])
    for a in range(NECH):
        pa = a % 2
        pltpu.make_async_copy(src_ref.at[pl.ds(ebase + a * ECH, ECH)],
                              esrcs[pa], esems[pa]).wait()
        pltpu.make_async_copy(dst_ref.at[pl.ds(ebase + a * ECH, ECH)],
                              edsts[pa], esems[pa]).wait()
        if a + 1 < NECH:
            pn = (a + 1) % 2
            pltpu.async_copy(src_ref.at[pl.ds(ebase + (a + 1) * ECH, ECH)],
                             esrcs[pn], esems[pn])
            pltpu.async_copy(dst_ref.at[pl.ds(ebase + (a + 1) * ECH, ECH)],
                             edsts[pn], esems[pn])
        esrc_v = esrcs[pa]
        edst_v = edsts[pa]

        def _bucket(j, state):
            cur, fil = state
            s16 = esrc_v[pl.ds(j * 16, 16)]
            d16 = edst_v[pl.ds(j * 16, 16)]
            rh16 = (d16 >= HALF).astype(jnp.int32)
            loc16 = d16 - rh16 * HALF
            pk16 = s16 * PKS + loc16
            part = plsc.load_gather(pmap_v, [s16])
            ncur = []
            nfil = []
            for b in range(NB):
                s, rh = b // 2, b % 2
                m = (part == s) & (rh16 == rh)
                n = jnp.sum(m.astype(jnp.int32))
                plsc.store_compressed(sts[b].at[pl.ds(fil[b], 16)],
                                      pk16, mask=m)
                c2 = cur[b] + n
                f2 = fil[b] + n
                flushed = c2 - f2

                @pl.when(f2 >= CHUNK)
                def _():
                    off = pl.multiple_of(
                        (b * NTILES + wid) * CAP + flushed, 8)
                    pltpu.sync_copy(sts[b].at[pl.ds(0, CHUNK)],
                                    bkt_ref.at[pl.ds(off, CHUNK)])
                    rem = sts[b][pl.ds(CHUNK, 16)]
                    sts[b][pl.ds(0, 16)] = rem

                f2 = jnp.where(f2 >= CHUNK, f2 - CHUNK, f2)
                ncur.append(c2)
                nfil.append(f2)
            return tuple(ncur), tuple(nfil)
        cur, fil = lax.fori_loop(0, ECH // 16, _bucket, (cur, fil))

    for b in range(NB):
        for k in range(CHUNK // 16):
            pos = k * 16 + lanes
            stv = sts[b][pl.ds(k * 16, 16)]
            sts[b][pl.ds(k * 16, 16)] = jnp.where(pos >= fil[b], dum_pk, stv)
        flushed = cur[b] - fil[b]

        @pl.when(fil[b] > 0)
        def _():
            off = pl.multiple_of((b * NTILES + wid) * CAP + flushed, 8)
            pltpu.sync_copy(sts[b].at[pl.ds(0, CHUNK)],
                            bkt_ref.at[pl.ds(off, CHUNK)])

    nch = [(cur[b] + (CHUNK - 1)) // CHUNK for b in range(NB)]

    srows = HALF // 16
    dump_sl = pl.ds(sid * srows, srows)

    def _dump_copy(b):
        s, rh = b // 2, b % 2
        return pltpu.make_async_copy(
            acc.at[dump_sl, :],
            ssum_ref.at[2 * s + cid,
                        pl.ds(rh * HALF + sid * srows, srows), :],
            sem_eb)

    for b in range(NB):
        if b > 0:
            _dump_copy(b - 1).wait()
        pltpu.async_copy(z2d, acc.at[pl.ds(sid * srows, CHUNK), :], sem_g0)
        pltpu.async_copy(z2d, acc.at[pl.ds(sid * srows + CHUNK, CHUNK), :],
                         sem_g1)
        pltpu.async_copy(z2d.at[pl.ds(0, srows - 2 * CHUNK), :],
                         acc.at[pl.ds(sid * srows + 2 * CHUNK,
                                      srows - 2 * CHUNK), :], sem_ea)
        pltpu.make_async_copy(
            z2d, acc.at[pl.ds(sid * srows, CHUNK), :], sem_g0).wait()
        pltpu.make_async_copy(
            z2d, acc.at[pl.ds(sid * srows + CHUNK, CHUNK), :], sem_g1).wait()
        pltpu.make_async_copy(
            z2d.at[pl.ds(0, srows - 2 * CHUNK), :],
            acc.at[pl.ds(sid * srows + 2 * CHUNK,
                         srows - 2 * CHUNK), :], sem_ea).wait()
        plsc.subcore_barrier()
        s, rh = b // 2, b % 2

        cbase = s * NPAD + rh * HALF
        bkbase = (b * NTILES + wid) * CAP

        def _pk_copy(c, p):
            off = pl.multiple_of(bkbase + c * CHUNK, 8)
            return pltpu.make_async_copy(bkt_ref.at[pl.ds(off, CHUNK)],
                                         pkbufs[p], sem_pk[p])

        for p in range(2):
            @pl.when(p < nch[b])
            def _(p=p):
                off = pl.multiple_of(bkbase + p * CHUNK, 8)
                pltpu.async_copy(bkt_ref.at[pl.ds(off, CHUNK)],
                                 pkbufs[p], sem_pk[p])

        def _pair(i, _):
            for p in range(2):
                c = 2 * i + p

                @pl.when(c < nch[b])
                def _(c=c, p=p):
                    @pl.when(c >= 2)
                    def _():
                        pltpu.make_async_copy(
                            rowbufs[p], acc.at[stages[p].at[0]],
                            sem_s[p]).wait()
                        pltpu.make_async_copy(
                            ones_v, cntacc.at[stages[p].at[2]],
                            sem_c[p]).wait()
                    _pk_copy(c, p).wait()
                    for k in range(CHUNK // 16):
                        pk = pkbufs[p][pl.ds(k * 16, 16)]
                        loc = pk & (PKS - 1)
                        stages[p][0, pl.ds(k * 16, 16)] = loc
                        stages[p][1, pl.ds(k * 16, 16)] = pk >> 14
                        stages[p][2, pl.ds(k * 16, 16)] = jnp.where(
                            loc >= HALF, P * NPAD + (loc - HALF), cbase + loc)

                    @pl.when(c + 2 < nch[b])
                    def _():
                        off2 = pl.multiple_of(bkbase + (c + 2) * CHUNK, 8)
                        pltpu.async_copy(bkt_ref.at[pl.ds(off2, CHUNK)],
                                         pkbufs[p], sem_pk[p])
                    pltpu.async_copy(x_ref.at[stages[p].at[1]],
                                     rowbufs[p], sem_g[p])
            for p in range(2):
                c = 2 * i + p

                @pl.when(c < nch[b])
                def _(c=c, p=p):
                    pltpu.make_async_copy(x_ref.at[stages[p].at[1]],
                                          rowbufs[p], sem_g[p]).wait()
                    pltpu.async_copy(rowbufs[p], acc.at[stages[p].at[0]],
                                     sem_s[p], add=True)
                    pltpu.async_copy(ones_v, cntacc.at[stages[p].at[2]],
                                     sem_c[p], add=True)
            return 0
        lax.fori_loop(0, (nch[b] + 1) // 2, _pair, 0)
        for p in range(2):
            @pl.when(nch[b] > p)
            def _(p=p):
                pltpu.make_async_copy(rowbufs[p], acc.at[stages[p].at[0]],
                                      sem_s[p]).wait()
                pltpu.make_async_copy(ones_v, cntacc.at[stages[p].at[2]],
                                      sem_c[p]).wait()
        plsc.subcore_barrier()

        _dump_copy(b).start()

    _dump_copy(NB - 1).wait()

    coff = pl.multiple_of(cid * (P * NPAD) + sid * (P * NPAD // 16), 8)
    pltpu.sync_copy(
        cntacc.at[pl.ds(pl.multiple_of(sid * (P * NPAD // 16), 8),
                        P * NPAD // 16)],
        cnt_ref.at[pl.ds(coff, P * NPAD // 16)])


@jax.jit
def _sc_aggregate(x, src, dst, p_map):
    mesh = plsc.VectorSubcoreMesh(core_axis_name="c", subcore_axis_name="s")
    f = pl.kernel(
        _sc_body,
        out_type=(
            jax.ShapeDtypeStruct((NB, NPAD, D), jnp.float32),
            jax.ShapeDtypeStruct((2 * P * NPAD,), jnp.float32),
            jax.ShapeDtypeStruct((NB * NTILES * CAP,), jnp.int32),
        ),
        mesh=mesh,
        compiler_params=pltpu.CompilerParams(needs_layout_passes=False),
        scratch_types=[
            pltpu.VMEM((N,), jnp.int32),
            pltpu.VMEM((STG,), jnp.int32),
            pltpu.VMEM((STG,), jnp.int32),
            pltpu.VMEM((STG,), jnp.int32),
            pltpu.VMEM((STG,), jnp.int32),
            pltpu.VMEM((STG,), jnp.int32),
            pltpu.VMEM((STG,), jnp.int32),
            pltpu.VMEM((STG,), jnp.int32),
            pltpu.VMEM((STG,), jnp.int32),
            pltpu.VMEM((ECH,), jnp.int32),
            pltpu.VMEM((ECH,), jnp.int32),
            pltpu.VMEM((ECH,), jnp.int32),
            pltpu.VMEM((ECH,), jnp.int32),
            pltpu.VMEM((CHUNK, D), jnp.float32),
            pltpu.VMEM((CHUNK, D), jnp.float32),
            pltpu.VMEM((CHUNK,), jnp.int32),
            pltpu.VMEM((CHUNK,), jnp.int32),
            pltpu.VMEM((3, CHUNK), jnp.int32),
            pltpu.VMEM((3, CHUNK), jnp.int32),
            pltpu.VMEM((CHUNK,), jnp.float32),
            pltpu.VMEM((CHUNK, D), jnp.float32),
            pltpu.VMEM((CSIZE // 16,), jnp.float32),
            pltpu.SemaphoreType.DMA,
            pltpu.SemaphoreType.DMA,
            pltpu.SemaphoreType.DMA,
            pltpu.SemaphoreType.DMA,
            pltpu.SemaphoreType.DMA,
            pltpu.SemaphoreType.DMA,
            pltpu.SemaphoreType.DMA,
            pltpu.SemaphoreType.DMA,
            pltpu.SemaphoreType.DMA,
            pltpu.SemaphoreType.DMA,
            pltpu.VMEM_SHARED((ACCR, D), jnp.float32),
            pltpu.VMEM_SHARED((CSIZE,), jnp.float32),
        ],
    )
    return f(x, src, dst, p_map)


def _tc_body(ssum_ref, cnt_ref, x_ref, pmap_ref, ws_ref, wn_ref, b_ref,
             out_ref):
    xb = x_ref[...]
    out = jnp.zeros_like(out_ref)
    for s in range(P):
        ssb = ssum_ref[2 * s] + ssum_ref[2 * s + 1]
        c = cnt_ref[s] + cnt_ref[P + s]
        inv = 1.0 / jnp.maximum(c, 1.0)
        mean = ssb * inv[:, None]
        out += lax.dot_general(mean, wn_ref[s], (((1,), (0,)), ((), ())),
                               preferred_element_type=jnp.float32,
                               precision=lax.Precision.HIGHEST)
    for t in range(P):
        sel = (pmap_ref[...] == t).astype(jnp.float32)
        h = lax.dot_general(xb, ws_ref[t], (((1,), (0,)), ((), ())),
                            preferred_element_type=jnp.float32,
                            precision=lax.Precision.HIGHEST)
        out += sel * (h + b_ref[t][None, :])
    out_ref[...] = out


@jax.jit
def _tc_merge(ssum, cnt_r, x_pad, pmap_b, W_self, W_neigh, b_pad):
    BLK = 1024
    grid = NPAD // BLK
    return pl.pallas_call(
        _tc_body,
        grid=(grid,),
        in_specs=[
            pl.BlockSpec((NB, BLK, D), lambda i: (0, i, 0)),
            pl.BlockSpec((2 * P, BLK), lambda i: (0, i)),
            pl.BlockSpec((BLK, D), lambda i: (i, 0)),
            pl.BlockSpec((BLK, D), lambda i: (i, 0)),
            pl.BlockSpec((P, D, D), lambda i: (0, 0, 0)),
            pl.BlockSpec((P, D, D), lambda i: (0, 0, 0)),
            pl.BlockSpec((2 * P, D), lambda i: (0, 0)),
        ],
        out_specs=pl.BlockSpec((BLK, D), lambda i: (i, 0)),
        out_shape=jax.ShapeDtypeStruct((NPAD, D), jnp.float32),
    )(ssum, cnt_r, x_pad, pmap_b, W_self, W_neigh, b_pad)


def kernel(x, edge_index, p_map, W_self, W_neigh, b):
    src = edge_index[0]
    dst = edge_index[1]
    ssum, cnt, _ = _sc_aggregate(x, src, dst, p_map)
    cnt_r = cnt.reshape(2 * P, NPAD)
    x_pad = jnp.pad(x, ((0, NPAD - N), (0, 0)))
    pmap_b = jnp.broadcast_to(jnp.pad(p_map, (0, NPAD - N))[:, None],
                              (NPAD, D))
    b_pad = jnp.pad(b, ((0, P), (0, 0)))
    out = _tc_merge(ssum, cnt_r, x_pad, pmap_b, W_self, W_neigh, b_pad)
    return out[:N]


# speedup vs baseline: 1.1346x; 1.0576x over previous
import jax
import jax.numpy as jnp
from jax import lax
from jax.experimental import pallas as pl
from jax.experimental.pallas import tpu as pltpu
from jax.experimental.pallas import tpu_sc as plsc

N = 10000
E = 320000
D = 128
P = 4
NPAD = 10240
HALF = NPAD // 2
NB = 2 * P
NTILES = 32
EPT = E // NTILES
ECH = 2000
NECH = EPT // ECH
CAP = 10240
CHUNK = 128
STG = CHUNK + 16
DROW = 64
ACCR = HALF + 2 * DROW
CSIZE = P * NPAD + 256
PKS = 16384


def _sc_body(x_ref, src_ref, dst_ref, pmap_ref,
             ssum_ref, cnt_ref, bkt_ref,
             pmap_v,
             st0, st1, st2, st3, st4, st5, st6, st7,
             esrc_a, esrc_b, edst_a, edst_b,
             rowbuf0, rowbuf1, pkbuf0, pkbuf1,
             stage0, stage1, ones_v, z2d, z1d,
             sem_ea, sem_eb, sem_pk0, sem_pk1,
             sem_g0, sem_g1, sem_s0, sem_s1, sem_c0, sem_c1,
             acc, cntacc):
    sts = [st0, st1, st2, st3, st4, st5, st6, st7]
    esrcs, edsts = [esrc_a, esrc_b], [edst_a, edst_b]
    esems = [sem_ea, sem_eb]
    rowbufs, pkbufs, stages = [rowbuf0, rowbuf1], [pkbuf0, pkbuf1], [stage0, stage1]
    sem_pk, sem_g, sem_s, sem_c = ([sem_pk0, sem_pk1], [sem_g0, sem_g1],
                                   [sem_s0, sem_s1], [sem_c0, sem_c1])
    cid = lax.axis_index("c")
    sid = lax.axis_index("s")
    wid = cid * 16 + sid
    ebase = pl.multiple_of(wid * EPT, 8)
    lanes = lax.iota(jnp.int32, 16)

    pltpu.sync_copy(pmap_ref, pmap_v)

    ones16 = jnp.ones((16,), jnp.float32)
    for k in range(CHUNK // 16):
        ones_v[pl.ds(k * 16, 16)] = ones16

    z16f = jnp.zeros((16,), jnp.float32)

    def _zero_z2d(j, _):
        r = j // (D // 16)
        c = (j % (D // 16)) * 16
        z2d[r, pl.ds(c, 16)] = z16f
        return 0
    lax.fori_loop(0, CHUNK * (D // 16), _zero_z2d, 0)

    def _zero_z1d(j, _):
        z1d[pl.ds(j * 16, 16)] = z16f
        return 0
    lax.fori_loop(0, (CSIZE // 16) // 16, _zero_z1d, 0)

    pltpu.sync_copy(z1d, cntacc.at[pl.ds(
        pl.multiple_of(sid * (CSIZE // 16), 8), CSIZE // 16)])

    dum_src = (wid * 313 + lanes * 13) % N
    dum_loc = HALF + (wid * 16 + lanes) % DROW
    dum_pk = dum_src * PKS + dum_loc

    zero8 = tuple(jnp.int32(0) for _ in range(NB))
    cur = zero8
    fil = zero8
    pltpu.async_copy(src_ref.at[pl.ds(ebase, ECH)], esrcs[0], esems[0])
    pltpu.async_copy(dst_ref.at[pl.ds(ebase, ECH)], edsts[0], esems[0])
    for a in range(NECH):
        pa = a % 2
        pltpu.make_async_copy(src_ref.at[pl.ds(ebase + a * ECH, ECH)],
                              esrcs[pa], esems[pa]).wait()
        pltpu.make_async_copy(dst_ref.at[pl.ds(ebase + a * ECH, ECH)],
                              edsts[pa], esems[pa]).wait()
        if a + 1 < NECH:
            pn = (a + 1) % 2
            pltpu.async_copy(src_ref.at[pl.ds(ebase + (a + 1) * ECH, ECH)],
                             esrcs[pn], esems[pn])
            pltpu.async_copy(dst_ref.at[pl.ds(ebase + (a + 1) * ECH, ECH)],
                             edsts[pn], esems[pn])
        esrc_v = esrcs[pa]
        edst_v = edsts[pa]

        def _bucket(j, state):
            cur, fil = state
            s16 = esrc_v[pl.ds(j * 16, 16)]
            d16 = edst_v[pl.ds(j * 16, 16)]
            rh16 = (d16 >= HALF).astype(jnp.int32)
            loc16 = d16 - rh16 * HALF
            pk16 = s16 * PKS + loc16
            part = plsc.load_gather(pmap_v, [s16])
            ncur = []
            nfil = []
            for b in range(NB):
                s, rh = b // 2, b % 2
                m = (part == s) & (rh16 == rh)
                n = jnp.sum(m.astype(jnp.int32))
                plsc.store_compressed(sts[b].at[pl.ds(fil[b], 16)],
                                      pk16, mask=m)
                c2 = cur[b] + n
                f2 = fil[b] + n
                flushed = c2 - f2

                @pl.when(f2 >= CHUNK)
                def _():
                    off = pl.multiple_of(
                        (b * NTILES + wid) * CAP + flushed, 8)
                    pltpu.sync_copy(sts[b].at[pl.ds(0, CHUNK)],
                                    bkt_ref.at[pl.ds(off, CHUNK)])
                    rem = sts[b][pl.ds(CHUNK, 16)]
                    sts[b][pl.ds(0, 16)] = rem

                f2 = jnp.where(f2 >= CHUNK, f2 - CHUNK, f2)
                ncur.append(c2)
                nfil.append(f2)
            return tuple(ncur), tuple(nfil)
        cur, fil = lax.fori_loop(0, ECH // 16, _bucket, (cur, fil))

    for b in range(NB):
        for k in range(CHUNK // 16):
            pos = k * 16 + lanes
            stv = sts[b][pl.ds(k * 16, 16)]
            sts[b][pl.ds(k * 16, 16)] = jnp.where(pos >= fil[b], dum_pk, stv)
        flushed = cur[b] - fil[b]

        @pl.when(fil[b] > 0)
        def _():
            off = pl.multiple_of((b * NTILES + wid) * CAP + flushed, 8)
            pltpu.sync_copy(sts[b].at[pl.ds(0, CHUNK)],
                            bkt_ref.at[pl.ds(off, CHUNK)])

    nch = [(cur[b] + (CHUNK - 1)) // CHUNK for b in range(NB)]

    srows = HALF // 16
    dump_sl = pl.ds(sid * srows, srows)

    def _dump_copy(b):
        s, rh = b // 2, b % 2
        return pltpu.make_async_copy(
            acc.at[dump_sl, :],
            ssum_ref.at[2 * s + cid,
                        pl.ds(rh * HALF + sid * srows, srows), :],
            sem_eb)

    for b in range(NB):
        if b > 0:
            _dump_copy(b - 1).wait()
        pltpu.async_copy(z2d, acc.at[pl.ds(sid * srows, CHUNK), :], sem_g0)
        pltpu.async_copy(z2d, acc.at[pl.ds(sid * srows + CHUNK, CHUNK), :],
                         sem_g1)
        pltpu.async_copy(z2d.at[pl.ds(0, srows - 2 * CHUNK), :],
                         acc.at[pl.ds(sid * srows + 2 * CHUNK,
                                      srows - 2 * CHUNK), :], sem_ea)
        pltpu.make_async_copy(
            z2d, acc.at[pl.ds(sid * srows, CHUNK), :], sem_g0).wait()
        pltpu.make_async_copy(
            z2d, acc.at[pl.ds(sid * srows + CHUNK, CHUNK), :], sem_g1).wait()
        pltpu.make_async_copy(
            z2d.at[pl.ds(0, srows - 2 * CHUNK), :],
            acc.at[pl.ds(sid * srows + 2 * CHUNK,
                         srows - 2 * CHUNK), :], sem_ea).wait()
        plsc.subcore_barrier()
        s, rh = b // 2, b % 2

        cbase = s * NPAD + rh * HALF
        bkbase = (b * NTILES + wid) * CAP

        def _pk_copy(c, p):
            off = pl.multiple_of(bkbase + c * CHUNK, 8)
            return pltpu.make_async_copy(bkt_ref.at[pl.ds(off, CHUNK)],
                                         pkbufs[p], sem_pk[p])

        for p in range(2):
            @pl.when(p < nch[b])
            def _(p=p):
                off = pl.multiple_of(bkbase + p * CHUNK, 8)
                pltpu.async_copy(bkt_ref.at[pl.ds(off, CHUNK)],
                                 pkbufs[p], sem_pk[p])

        def _pair(i, _):
            for p in range(2):
                c = 2 * i + p

                @pl.when(c < nch[b])
                def _(c=c, p=p):
                    @pl.when(c >= 2)
                    def _():
                        pltpu.make_async_copy(
                            rowbufs[p], acc.at[stages[p].at[0]],
                            sem_s[p]).wait()
                        pltpu.make_async_copy(
                            ones_v, cntacc.at[stages[p].at[2]],
                            sem_c[p]).wait()
                    _pk_copy(c, p).wait()
                    for k in range(CHUNK // 16):
                        pk = pkbufs[p][pl.ds(k * 16, 16)]
                        loc = pk & (PKS - 1)
                        stages[p][0, pl.ds(k * 16, 16)] = loc
                        stages[p][1, pl.ds(k * 16, 16)] = pk >> 14
                        stages[p][2, pl.ds(k * 16, 16)] = jnp.where(
                            loc >= HALF, P * NPAD + (loc - HALF), cbase + loc)

                    @pl.when(c + 2 < nch[b])
                    def _():
                        off2 = pl.multiple_of(bkbase + (c + 2) * CHUNK, 8)
                        pltpu.async_copy(bkt_ref.at[pl.ds(off2, CHUNK)],
                                         pkbufs[p], sem_pk[p])
                    pltpu.async_copy(x_ref.at[stages[p].at[1]],
                                     rowbufs[p], sem_g[p])
            for p in range(2):
                c = 2 * i + p

                @pl.when(c < nch[b])
                def _(c=c, p=p):
                    pltpu.make_async_copy(x_ref.at[stages[p].at[1]],
                                          rowbufs[p], sem_g[p]).wait()
                    pltpu.async_copy(rowbufs[p], acc.at[stages[p].at[0]],
                                     sem_s[p], add=True)
                    pltpu.async_copy(ones_v, cntacc.at[stages[p].at[2]],
                                     sem_c[p], add=True)
            return 0
        lax.fori_loop(0, (nch[b] + 1) // 2, _pair, 0)
        for p in range(2):
            @pl.when(nch[b] > p)
            def _(p=p):
                pltpu.make_async_copy(rowbufs[p], acc.at[stages[p].at[0]],
                                      sem_s[p]).wait()
                pltpu.make_async_copy(ones_v, cntacc.at[stages[p].at[2]],
                                      sem_c[p]).wait()
        plsc.subcore_barrier()

        _dump_copy(b).start()

    _dump_copy(NB - 1).wait()

    coff = pl.multiple_of(cid * (P * NPAD) + sid * (P * NPAD // 16), 8)
    pltpu.sync_copy(
        cntacc.at[pl.ds(pl.multiple_of(sid * (P * NPAD // 16), 8),
                        P * NPAD // 16)],
        cnt_ref.at[pl.ds(coff, P * NPAD // 16)])


@jax.jit
def _sc_aggregate(x, src, dst, p_map):
    mesh = plsc.VectorSubcoreMesh(core_axis_name="c", subcore_axis_name="s")
    f = pl.kernel(
        _sc_body,
        out_type=(
            jax.ShapeDtypeStruct((NB, NPAD, D), jnp.float32),
            jax.ShapeDtypeStruct((2 * P * NPAD,), jnp.float32),
            jax.ShapeDtypeStruct((NB * NTILES * CAP,), jnp.int32),
        ),
        mesh=mesh,
        compiler_params=pltpu.CompilerParams(needs_layout_passes=False),
        scratch_types=[
            pltpu.VMEM((N,), jnp.int32),
            pltpu.VMEM((STG,), jnp.int32),
            pltpu.VMEM((STG,), jnp.int32),
            pltpu.VMEM((STG,), jnp.int32),
            pltpu.VMEM((STG,), jnp.int32),
            pltpu.VMEM((STG,), jnp.int32),
            pltpu.VMEM((STG,), jnp.int32),
            pltpu.VMEM((STG,), jnp.int32),
            pltpu.VMEM((STG,), jnp.int32),
            pltpu.VMEM((ECH,), jnp.int32),
            pltpu.VMEM((ECH,), jnp.int32),
            pltpu.VMEM((ECH,), jnp.int32),
            pltpu.VMEM((ECH,), jnp.int32),
            pltpu.VMEM((CHUNK, D), jnp.float32),
            pltpu.VMEM((CHUNK, D), jnp.float32),
            pltpu.VMEM((CHUNK,), jnp.int32),
            pltpu.VMEM((CHUNK,), jnp.int32),
            pltpu.VMEM((3, CHUNK), jnp.int32),
            pltpu.VMEM((3, CHUNK), jnp.int32),
            pltpu.VMEM((CHUNK,), jnp.float32),
            pltpu.VMEM((CHUNK, D), jnp.float32),
            pltpu.VMEM((CSIZE // 16,), jnp.float32),
            pltpu.SemaphoreType.DMA,
            pltpu.SemaphoreType.DMA,
            pltpu.SemaphoreType.DMA,
            pltpu.SemaphoreType.DMA,
            pltpu.SemaphoreType.DMA,
            pltpu.SemaphoreType.DMA,
            pltpu.SemaphoreType.DMA,
            pltpu.SemaphoreType.DMA,
            pltpu.SemaphoreType.DMA,
            pltpu.SemaphoreType.DMA,
            pltpu.VMEM_SHARED((ACCR, D), jnp.float32),
            pltpu.VMEM_SHARED((CSIZE,), jnp.float32),
        ],
    )
    return f(x, src, dst, p_map)


BLK = 1024


def _tc_self_body(x_ref, pmap_ref, ws_ref, b_ref, out_ref):
    xb = x_ref[...]
    out = jnp.zeros_like(out_ref)
    for t in range(P):
        sel = (pmap_ref[...] == t).astype(jnp.float32)
        h = lax.dot_general(xb, ws_ref[t], (((1,), (0,)), ((), ())),
                            preferred_element_type=jnp.float32,
                            precision=lax.Precision.HIGHEST)
        out += sel * (h + b_ref[t][None, :])
    out_ref[...] = out


@jax.jit
def _tc_self(x_pad, pmap_b, W_self, b_pad):
    return pl.pallas_call(
        _tc_self_body,
        grid=(NPAD // BLK,),
        in_specs=[
            pl.BlockSpec((BLK, D), lambda i: (i, 0)),
            pl.BlockSpec((BLK, D), lambda i: (i, 0)),
            pl.BlockSpec((P, D, D), lambda i: (0, 0, 0)),
            pl.BlockSpec((2 * P, D), lambda i: (0, 0)),
        ],
        out_specs=pl.BlockSpec((BLK, D), lambda i: (i, 0)),
        out_shape=jax.ShapeDtypeStruct((NPAD, D), jnp.float32),
    )(x_pad, pmap_b, W_self, b_pad)


def _tc_body(ssum_ref, cnt_ref, self_ref, wn_ref, out_ref):
    out = self_ref[...]
    for s in range(P):
        ssb = ssum_ref[2 * s] + ssum_ref[2 * s + 1]
        c = cnt_ref[s] + cnt_ref[P + s]
        inv = 1.0 / jnp.maximum(c, 1.0)
        mean = ssb * inv[:, None]
        out += lax.dot_general(mean, wn_ref[s], (((1,), (0,)), ((), ())),
                               preferred_element_type=jnp.float32,
                               precision=lax.Precision.HIGHEST)
    out_ref[...] = out


@jax.jit
def _tc_merge(ssum, cnt_r, selfh, W_neigh):
    return pl.pallas_call(
        _tc_body,
        grid=(NPAD // BLK,),
        in_specs=[
            pl.BlockSpec((NB, BLK, D), lambda i: (0, i, 0)),
            pl.BlockSpec((2 * P, BLK), lambda i: (0, i)),
            pl.BlockSpec((BLK, D), lambda i: (i, 0)),
            pl.BlockSpec((P, D, D), lambda i: (0, 0, 0)),
        ],
        out_specs=pl.BlockSpec((BLK, D), lambda i: (i, 0)),
        out_shape=jax.ShapeDtypeStruct((NPAD, D), jnp.float32),
    )(ssum, cnt_r, selfh, W_neigh)


def kernel(x, edge_index, p_map, W_self, W_neigh, b):
    src = edge_index[0]
    dst = edge_index[1]
    x_pad = jnp.pad(x, ((0, NPAD - N), (0, 0)))
    pmap_b = jnp.broadcast_to(jnp.pad(p_map, (0, NPAD - N))[:, None],
                              (NPAD, D))
    b_pad = jnp.pad(b, ((0, P), (0, 0)))
    selfh = _tc_self(x_pad, pmap_b, W_self, b_pad)
    ssum, cnt, _ = _sc_aggregate(x, src, dst, p_map)
    cnt_r = cnt.reshape(2 * P, NPAD)
    out = _tc_merge(ssum, cnt_r, selfh, W_neigh)
    return out[:N]


# speedup vs baseline: 1.1462x; 1.0103x over previous
import jax
import jax.numpy as jnp
from jax import lax
from jax.experimental import pallas as pl
from jax.experimental.pallas import tpu as pltpu
from jax.experimental.pallas import tpu_sc as plsc

N = 10000
E = 320000
D = 128
P = 4
NPAD = 10240
HALF = NPAD // 2
NB = 2 * P
NTILES = 32
EPT = E // NTILES
ECH = 2000
NECH = EPT // ECH
CAP = 10240
CHUNK = 128
STG = CHUNK + 16
DROW = 64
ACCR = HALF + 2 * DROW
CSIZE = P * NPAD + 256
PKS = 16384


def _sc_body(x_ref, src_ref, dst_ref, pmap_ref,
             ssum_ref, cnt_ref, bkt_ref,
             pmap_v,
             st0, st1, st2, st3, st4, st5, st6, st7,
             esrc_a, esrc_b, edst_a, edst_b,
             rowbuf0, rowbuf1, pkbuf0, pkbuf1,
             stage0, stage1, ones_v, z2d, z1d,
             sem_ea, sem_eb, sem_pk0, sem_pk1,
             sem_g0, sem_g1, sem_s0, sem_s1, sem_c0, sem_c1,
             acc, cntacc):
    sts = [st0, st1, st2, st3, st4, st5, st6, st7]
    esrcs, edsts = [esrc_a, esrc_b], [edst_a, edst_b]
    esems = [sem_ea, sem_eb]
    rowbufs, pkbufs, stages = [rowbuf0, rowbuf1], [pkbuf0, pkbuf1], [stage0, stage1]
    sem_pk, sem_g, sem_s, sem_c = ([sem_pk0, sem_pk1], [sem_g0, sem_g1],
                                   [sem_s0, sem_s1], [sem_c0, sem_c1])
    cid = lax.axis_index("c")
    sid = lax.axis_index("s")
    wid = cid * 16 + sid
    ebase = pl.multiple_of(wid * EPT, 8)
    lanes = lax.iota(jnp.int32, 16)

    pltpu.sync_copy(pmap_ref, pmap_v)

    ones16 = jnp.ones((16,), jnp.float32)
    for k in range(CHUNK // 16):
        ones_v[pl.ds(k * 16, 16)] = ones16

    z16f = jnp.zeros((16,), jnp.float32)

    def _zero_z2d(j, _):
        r = j // (D // 16)
        c = (j % (D // 16)) * 16
        z2d[r, pl.ds(c, 16)] = z16f
        return 0
    lax.fori_loop(0, CHUNK * (D // 16), _zero_z2d, 0)

    def _zero_z1d(j, _):
        z1d[pl.ds(j * 16, 16)] = z16f
        return 0
    lax.fori_loop(0, (CSIZE // 16) // 16, _zero_z1d, 0)

    pltpu.sync_copy(z1d, cntacc.at[pl.ds(
        pl.multiple_of(sid * (CSIZE // 16), 8), CSIZE // 16)])

    dum_src = (wid * 313 + lanes * 13) % N
    dum_loc = HALF + (wid * 16 + lanes) % DROW
    dum_pk = dum_src * PKS + dum_loc

    zero8 = tuple(jnp.int32(0) for _ in range(NB))
    cur = zero8
    fil = zero8
    pltpu.async_copy(src_ref.at[pl.ds(ebase, ECH)], esrcs[0], esems[0])
    pltpu.async_copy(dst_ref.at[pl.ds(ebase, ECH)], edsts[0], esems[0])
    for a in range(NECH):
        pa = a % 2
        pltpu.make_async_copy(src_ref.at[pl.ds(ebase + a * ECH, ECH)],
                              esrcs[pa], esems[pa]).wait()
        pltpu.make_async_copy(dst_ref.at[pl.ds(ebase + a * ECH, ECH)],
                              edsts[pa], esems[pa]).wait()
        if a + 1 < NECH:
            pn = (a + 1) % 2
            pltpu.async_copy(src_ref.at[pl.ds(ebase + (a + 1) * ECH, ECH)],
                             esrcs[pn], esems[pn])
            pltpu.async_copy(dst_ref.at[pl.ds(ebase + (a + 1) * ECH, ECH)],
                             edsts[pn], esems[pn])
        esrc_v = esrcs[pa]
        edst_v = edsts[pa]

        def _bucket(j, state):
            cur, fil = state
            s16 = esrc_v[pl.ds(j * 16, 16)]
            d16 = edst_v[pl.ds(j * 16, 16)]
            rh16 = (d16 >= HALF).astype(jnp.int32)
            loc16 = d16 - rh16 * HALF
            pk16 = s16 * PKS + loc16
            part = plsc.load_gather(pmap_v, [s16])
            ncur = []
            nfil = []
            for b in range(NB):
                s, rh = b // 2, b % 2
                m = (part == s) & (rh16 == rh)
                n = plsc.all_reduce_population_count(m)[0]
                plsc.store_compressed(sts[b].at[pl.ds(fil[b], 16)],
                                      pk16, mask=m)
                c2 = cur[b] + n
                f2 = fil[b] + n
                flushed = c2 - f2

                @pl.when(f2 >= CHUNK)
                def _():
                    off = pl.multiple_of(
                        (b * NTILES + wid) * CAP + flushed, 8)
                    pltpu.sync_copy(sts[b].at[pl.ds(0, CHUNK)],
                                    bkt_ref.at[pl.ds(off, CHUNK)])
                    rem = sts[b][pl.ds(CHUNK, 16)]
                    sts[b][pl.ds(0, 16)] = rem

                f2 = jnp.where(f2 >= CHUNK, f2 - CHUNK, f2)
                ncur.append(c2)
                nfil.append(f2)
            return tuple(ncur), tuple(nfil)
        cur, fil = lax.fori_loop(0, ECH // 16, _bucket, (cur, fil))

    for b in range(NB):
        for k in range(CHUNK // 16):
            pos = k * 16 + lanes
            stv = sts[b][pl.ds(k * 16, 16)]
            sts[b][pl.ds(k * 16, 16)] = jnp.where(pos >= fil[b], dum_pk, stv)
        flushed = cur[b] - fil[b]

        @pl.when(fil[b] > 0)
        def _():
            off = pl.multiple_of((b * NTILES + wid) * CAP + flushed, 8)
            pltpu.sync_copy(sts[b].at[pl.ds(0, CHUNK)],
                            bkt_ref.at[pl.ds(off, CHUNK)])

    nch = [(cur[b] + (CHUNK - 1)) // CHUNK for b in range(NB)]

    srows = HALF // 16
    dump_sl = pl.ds(sid * srows, srows)

    def _dump_copy(b):
        s, rh = b // 2, b % 2
        return pltpu.make_async_copy(
            acc.at[dump_sl, :],
            ssum_ref.at[2 * s + cid,
                        pl.ds(rh * HALF + sid * srows, srows), :],
            sem_eb)

    for b in range(NB):
        if b > 0:
            _dump_copy(b - 1).wait()
        pltpu.async_copy(z2d, acc.at[pl.ds(sid * srows, CHUNK), :], sem_g0)
        pltpu.async_copy(z2d, acc.at[pl.ds(sid * srows + CHUNK, CHUNK), :],
                         sem_g1)
        pltpu.async_copy(z2d.at[pl.ds(0, srows - 2 * CHUNK), :],
                         acc.at[pl.ds(sid * srows + 2 * CHUNK,
                                      srows - 2 * CHUNK), :], sem_ea)
        pltpu.make_async_copy(
            z2d, acc.at[pl.ds(sid * srows, CHUNK), :], sem_g0).wait()
        pltpu.make_async_copy(
            z2d, acc.at[pl.ds(sid * srows + CHUNK, CHUNK), :], sem_g1).wait()
        pltpu.make_async_copy(
            z2d.at[pl.ds(0, srows - 2 * CHUNK), :],
            acc.at[pl.ds(sid * srows + 2 * CHUNK,
                         srows - 2 * CHUNK), :], sem_ea).wait()
        plsc.subcore_barrier()
        s, rh = b // 2, b % 2

        cbase = s * NPAD + rh * HALF
        bkbase = (b * NTILES + wid) * CAP

        def _pk_copy(c, p):
            off = pl.multiple_of(bkbase + c * CHUNK, 8)
            return pltpu.make_async_copy(bkt_ref.at[pl.ds(off, CHUNK)],
                                         pkbufs[p], sem_pk[p])

        for p in range(2):
            @pl.when(p < nch[b])
            def _(p=p):
                off = pl.multiple_of(bkbase + p * CHUNK, 8)
                pltpu.async_copy(bkt_ref.at[pl.ds(off, CHUNK)],
                                 pkbufs[p], sem_pk[p])

        def _pair(i, _):
            for p in range(2):
                c = 2 * i + p

                @pl.when(c < nch[b])
                def _(c=c, p=p):
                    @pl.when(c >= 2)
                    def _():
                        pltpu.make_async_copy(
                            rowbufs[p], acc.at[stages[p].at[0]],
                            sem_s[p]).wait()
                        pltpu.make_async_copy(
                            ones_v, cntacc.at[stages[p].at[2]],
                            sem_c[p]).wait()
                    _pk_copy(c, p).wait()
                    for k in range(CHUNK // 16):
                        pk = pkbufs[p][pl.ds(k * 16, 16)]
                        loc = pk & (PKS - 1)
                        stages[p][0, pl.ds(k * 16, 16)] = loc
                        stages[p][1, pl.ds(k * 16, 16)] = pk >> 14
                        stages[p][2, pl.ds(k * 16, 16)] = jnp.where(
                            loc >= HALF, P * NPAD + (loc - HALF), cbase + loc)

                    @pl.when(c + 2 < nch[b])
                    def _():
                        off2 = pl.multiple_of(bkbase + (c + 2) * CHUNK, 8)
                        pltpu.async_copy(bkt_ref.at[pl.ds(off2, CHUNK)],
                                         pkbufs[p], sem_pk[p])
                    pltpu.async_copy(x_ref.at[stages[p].at[1]],
                                     rowbufs[p], sem_g[p])
            for p in range(2):
                c = 2 * i + p

                @pl.when(c < nch[b])
                def _(c=c, p=p):
                    pltpu.make_async_copy(x_ref.at[stages[p].at[1]],
                                          rowbufs[p], sem_g[p]).wait()
                    pltpu.async_copy(rowbufs[p], acc.at[stages[p].at[0]],
                                     sem_s[p], add=True)
                    pltpu.async_copy(ones_v, cntacc.at[stages[p].at[2]],
                                     sem_c[p], add=True)
            return 0
        lax.fori_loop(0, (nch[b] + 1) // 2, _pair, 0)
        for p in range(2):
            @pl.when(nch[b] > p)
            def _(p=p):
                pltpu.make_async_copy(rowbufs[p], acc.at[stages[p].at[0]],
                                      sem_s[p]).wait()
                pltpu.make_async_copy(ones_v, cntacc.at[stages[p].at[2]],
                                      sem_c[p]).wait()
        plsc.subcore_barrier()

        _dump_copy(b).start()

    _dump_copy(NB - 1).wait()

    coff = pl.multiple_of(cid * (P * NPAD) + sid * (P * NPAD // 16), 8)
    pltpu.sync_copy(
        cntacc.at[pl.ds(pl.multiple_of(sid * (P * NPAD // 16), 8),
                        P * NPAD // 16)],
        cnt_ref.at[pl.ds(coff, P * NPAD // 16)])


@jax.jit
def _sc_aggregate(x, src, dst, p_map):
    mesh = plsc.VectorSubcoreMesh(core_axis_name="c", subcore_axis_name="s")
    f = pl.kernel(
        _sc_body,
        out_type=(
            jax.ShapeDtypeStruct((NB, NPAD, D), jnp.float32),
            jax.ShapeDtypeStruct((2 * P * NPAD,), jnp.float32),
            jax.ShapeDtypeStruct((NB * NTILES * CAP,), jnp.int32),
        ),
        mesh=mesh,
        compiler_params=pltpu.CompilerParams(needs_layout_passes=False),
        scratch_types=[
            pltpu.VMEM((N,), jnp.int32),
            pltpu.VMEM((STG,), jnp.int32),
            pltpu.VMEM((STG,), jnp.int32),
            pltpu.VMEM((STG,), jnp.int32),
            pltpu.VMEM((STG,), jnp.int32),
            pltpu.VMEM((STG,), jnp.int32),
            pltpu.VMEM((STG,), jnp.int32),
            pltpu.VMEM((STG,), jnp.int32),
            pltpu.VMEM((STG,), jnp.int32),
            pltpu.VMEM((ECH,), jnp.int32),
            pltpu.VMEM((ECH,), jnp.int32),
            pltpu.VMEM((ECH,), jnp.int32),
            pltpu.VMEM((ECH,), jnp.int32),
            pltpu.VMEM((CHUNK, D), jnp.float32),
            pltpu.VMEM((CHUNK, D), jnp.float32),
            pltpu.VMEM((CHUNK,), jnp.int32),
            pltpu.VMEM((CHUNK,), jnp.int32),
            pltpu.VMEM((3, CHUNK), jnp.int32),
            pltpu.VMEM((3, CHUNK), jnp.int32),
            pltpu.VMEM((CHUNK,), jnp.float32),
            pltpu.VMEM((CHUNK, D), jnp.float32),
            pltpu.VMEM((CSIZE // 16,), jnp.float32),
            pltpu.SemaphoreType.DMA,
            pltpu.SemaphoreType.DMA,
            pltpu.SemaphoreType.DMA,
            pltpu.SemaphoreType.DMA,
            pltpu.SemaphoreType.DMA,
            pltpu.SemaphoreType.DMA,
            pltpu.SemaphoreType.DMA,
            pltpu.SemaphoreType.DMA,
            pltpu.SemaphoreType.DMA,
            pltpu.SemaphoreType.DMA,
            pltpu.VMEM_SHARED((ACCR, D), jnp.float32),
            pltpu.VMEM_SHARED((CSIZE,), jnp.float32),
        ],
    )
    return f(x, src, dst, p_map)


BLK = 1024


def _tc_self_body(x_ref, pmap_ref, ws_ref, b_ref, out_ref):
    xb = x_ref[...]
    out = jnp.zeros_like(out_ref)
    for t in range(P):
        sel = (pmap_ref[...] == t).astype(jnp.float32)
        h = lax.dot_general(xb, ws_ref[t], (((1,), (0,)), ((), ())),
                            preferred_element_type=jnp.float32,
                            precision=lax.Precision.HIGHEST)
        out += sel * (h + b_ref[t][None, :])
    out_ref[...] = out


@jax.jit
def _tc_self(x_pad, pmap_b, W_self, b_pad):
    return pl.pallas_call(
        _tc_self_body,
        grid=(NPAD // BLK,),
        in_specs=[
            pl.BlockSpec((BLK, D), lambda i: (i, 0)),
            pl.BlockSpec((BLK, D), lambda i: (i, 0)),
            pl.BlockSpec((P, D, D), lambda i: (0, 0, 0)),
            pl.BlockSpec((2 * P, D), lambda i: (0, 0)),
        ],
        out_specs=pl.BlockSpec((BLK, D), lambda i: (i, 0)),
        out_shape=jax.ShapeDtypeStruct((NPAD, D), jnp.float32),
    )(x_pad, pmap_b, W_self, b_pad)


def _tc_body(ssum_ref, cnt_ref, self_ref, wn_ref, out_ref):
    out = self_ref[...]
    for s in range(P):
        ssb = ssum_ref[2 * s] + ssum_ref[2 * s + 1]
        c = cnt_ref[s] + cnt_ref[P + s]
        inv = 1.0 / jnp.maximum(c, 1.0)
        mean = ssb * inv[:, None]
        out += lax.dot_general(mean, wn_ref[s], (((1,), (0,)), ((), ())),
                               preferred_element_type=jnp.float32,
                               precision=lax.Precision.HIGHEST)
    out_ref[...] = out


@jax.jit
def _tc_merge(ssum, cnt_r, selfh, W_neigh):
    return pl.pallas_call(
        _tc_body,
        grid=(NPAD // BLK,),
        in_specs=[
            pl.BlockSpec((NB, BLK, D), lambda i: (0, i, 0)),
            pl.BlockSpec((2 * P, BLK), lambda i: (0, i)),
            pl.BlockSpec((BLK, D), lambda i: (i, 0)),
            pl.BlockSpec((P, D, D), lambda i: (0, 0, 0)),
        ],
        out_specs=pl.BlockSpec((BLK, D), lambda i: (i, 0)),
        out_shape=jax.ShapeDtypeStruct((NPAD, D), jnp.float32),
    )(ssum, cnt_r, selfh, W_neigh)


def kernel(x, edge_index, p_map, W_self, W_neigh, b):
    src = edge_index[0]
    dst = edge_index[1]
    x_pad = jnp.pad(x, ((0, NPAD - N), (0, 0)))
    pmap_b = jnp.broadcast_to(jnp.pad(p_map, (0, NPAD - N))[:, None],
                              (NPAD, D))
    b_pad = jnp.pad(b, ((0, P), (0, 0)))
    selfh = _tc_self(x_pad, pmap_b, W_self, b_pad)
    ssum, cnt, _ = _sc_aggregate(x, src, dst, p_map)
    cnt_r = cnt.reshape(2 * P, NPAD)
    out = _tc_merge(ssum, cnt_r, selfh, W_neigh)
    return out[:N]
